# Initial kernel scaffold; baseline (speedup 1.0000x reference)
import jax
import jax.numpy as jnp
from jax.experimental import pallas as pl


def kernel(x, edge_index, batch, Wn0, bn0, Wr0, gamma0, beta0, Wn1, bn1, Wr1, gamma1, beta1, Wn2, bn2, Wr2, gamma2, beta2, Wc1, bc1, Wc2, bc2):
    raise NotImplementedError("write your pallas kernel here")


# speedup vs baseline: 3.0980x; 3.0980x over previous
import functools

import jax
import jax.numpy as jnp
from jax import lax
from jax.experimental import pallas as pl
from jax.experimental.pallas import tpu as pltpu
from jax.experimental.pallas import tpu_sc as plsc

N = 10000
E = 320000
D = 128
H = 256
G = 64

NC = 2
NS = 16
CH = 128
E_PAD = 327680
NACC = 10240
RPT = NACC // NS

NB = 1000
NBLK = N // NB

_MESH = plsc.VectorSubcoreMesh(core_axis_name="c", subcore_axis_name="s")


NCB = 80


def _make_segsum(nblocks):
  @functools.partial(
      pl.kernel,
      out_type=jax.ShapeDtypeStruct((NC, NACC, D), jnp.float32),
      mesh=_MESH,
      scratch_types=[
          pltpu.VMEM((NCB, CH), jnp.int32),
          pltpu.VMEM((NCB, CH), jnp.int32),
          pltpu.VMEM((CH, D), jnp.float32),
          pltpu.VMEM_SHARED((NACC, D), jnp.float32),
          pltpu.SemaphoreType.DMA,
      ],
  )
  def seg(table_hbm, src_hbm, dst_hbm, zeros_hbm, out_hbm,
          srcv, dstv, rows, acc, sem):
    cid = lax.axis_index("c")
    sid = lax.axis_index("s")
    tid = cid * NS + sid
    pltpu.sync_copy(zeros_hbm, acc.at[pl.ds(sid * RPT, RPT)])
    plsc.subcore_barrier()

    def blk(b, carry):
      pltpu.sync_copy(src_hbm.at[tid, pl.ds(b * NCB, NCB)], srcv)
      pltpu.sync_copy(dst_hbm.at[tid, pl.ds(b * NCB, NCB)], dstv)

      def body(g, c):
        pltpu.async_copy(table_hbm.at[srcv.at[g]], rows, sem).wait()
        pltpu.sync_copy(rows, acc.at[dstv.at[g]], add=True)
        return c

      lax.fori_loop(0, NCB, body, 0)
      return carry

    lax.fori_loop(0, nblocks, blk, 0)
    plsc.subcore_barrier()
    pltpu.sync_copy(acc.at[pl.ds(sid * RPT, RPT)],
                    out_hbm.at[cid, pl.ds(sid * RPT, RPT)])

  return seg


_SEG_A = _make_segsum(1)
_SEG_B = _make_segsum(2)


@functools.partial(
    pl.kernel,
    out_type=jax.ShapeDtypeStruct((NC, NACC, D), jnp.float32),
    mesh=_MESH,
    scratch_types=[
        pltpu.VMEM((E_PAD // 32 // CH, CH), jnp.int32),
        pltpu.VMEM((CH, D), jnp.float32),
        pltpu.VMEM_SHARED((NACC, D), jnp.float32),
    ],
)
def _cnt_kernel(dst_hbm, ones_hbm, zeros_hbm, out_hbm, dstv, onesv, acc):
  cid = lax.axis_index("c")
  sid = lax.axis_index("s")
  tid = cid * NS + sid
  pltpu.sync_copy(zeros_hbm, acc.at[pl.ds(sid * RPT, RPT)])
  pltpu.sync_copy(ones_hbm, onesv)
  pltpu.sync_copy(dst_hbm.at[tid], dstv)
  plsc.subcore_barrier()

  def body(g, carry):
    pltpu.sync_copy(onesv, acc.at[dstv.at[g]], add=True)
    return carry

  lax.fori_loop(0, E_PAD // 32 // CH, body, 0)
  plsc.subcore_barrier()
  pltpu.sync_copy(acc.at[pl.ds(sid * RPT, RPT)],
                  out_hbm.at[cid, pl.ds(sid * RPT, RPT)])


def _dot(a, b):
  return jnp.dot(a, b, preferred_element_type=jnp.float32)


def _c0_body(sp, cp, x, wn, wr, bn, pre, ssum, ssq):
  pid = pl.program_id(0)
  inv = 1.0 / jnp.maximum(cp[0] + cp[1], 1.0)
  s = (sp[0] + sp[1]) * inv
  acc = _dot(s, wn[...]) + _dot(x[...], wr[...]) + bn[...]
  pre[...] = acc

  @pl.when(pid == 0)
  def _():
    ssum[...] = jnp.zeros_like(ssum)
    ssq[...] = jnp.zeros_like(ssq)

  ssum[...] += jnp.sum(acc, axis=0, keepdims=True)
  ssq[...] += jnp.sum(acc * acc, axis=0, keepdims=True)


def _c_body(sp, cp, h, wn, wr, bn, pre, ssum, ssq):
  pid = pl.program_id(0)
  inv = 1.0 / jnp.maximum(cp[0] + cp[1], 1.0)
  w_n = wn[...]
  w_r = wr[...]
  acc = (_dot(sp[0] * inv, w_n[0:D]) + _dot(sp[1] * inv, w_n[D:2 * D])
         + _dot(h[0], w_r[0:D]) + _dot(h[1], w_r[D:2 * D]) + bn[...])
  pre[...] = acc

  @pl.when(pid == 0)
  def _():
    ssum[...] = jnp.zeros_like(ssum)
    ssq[...] = jnp.zeros_like(ssq)

  ssum[...] += jnp.sum(acc, axis=0, keepdims=True)
  ssq[...] += jnp.sum(acc * acc, axis=0, keepdims=True)


def _d_body(pre, ssum, ssq, gamma, beta, h_out):
  mu = ssum[...] * (1.0 / N)
  var = ssq[...] * (1.0 / N) - mu * mu
  scale = gamma[...] * lax.rsqrt(var + 1e-5)
  shift = beta[...] - mu * scale
  v = jnp.maximum(pre[...] * scale + shift, 0.0)
  h_out[0] = v[:, 0:D]
  h_out[1] = v[:, D:2 * D]


def _pool_body(h, bidx, wc1, bc1, wc2, bc2, out, gacc, ccnt):
  pid = pl.program_id(0)

  @pl.when(pid == 0)
  def _():
    gacc[...] = jnp.zeros_like(gacc)
    ccnt[...] = jnp.zeros_like(ccnt)

  onehot = (lax.broadcasted_iota(jnp.int32, (G, NB), 0)
            == bidx[0]).astype(jnp.float32)
  gacc[:, 0:D] += _dot(onehot, h[0])
  gacc[:, D:2 * D] += _dot(onehot, h[1])
  ccnt[...] += jnp.sum(onehot, axis=1, keepdims=True)

  @pl.when(pid == NBLK - 1)
  def _():
    inv = 1.0 / jnp.maximum(ccnt[...], 1.0)
    g = gacc[...] * jnp.concatenate([inv, inv], axis=1)
    z = jnp.maximum(_dot(g, wc1[...]) + bc1[...], 0.0)
    out[...] = jnp.sum(z * wc2[...], axis=1, keepdims=True) + bc2[...]


def _full(shape):
  return pl.BlockSpec(shape, lambda i: tuple(0 for _ in shape))


def _c0_call(sp, cp, x, wn, wr, bn):
  return pl.pallas_call(
      _c0_body,
      grid=(NBLK,),
      in_specs=[
          pl.BlockSpec((NC, NB, D), lambda i: (0, i, 0)),
          pl.BlockSpec((NC, NB, D), lambda i: (0, i, 0)),
          pl.BlockSpec((NB, D), lambda i: (i, 0)),
          _full((D, H)),
          _full((D, H)),
          _full((1, H)),
      ],
      out_specs=[
          pl.BlockSpec((NB, H), lambda i: (i, 0)),
          _full((1, H)),
          _full((1, H)),
      ],
      out_shape=[
          jax.ShapeDtypeStruct((N, H), jnp.float32),
          jax.ShapeDtypeStruct((1, H), jnp.float32),
          jax.ShapeDtypeStruct((1, H), jnp.float32),
      ],
  )(sp, cp, x, wn, wr, bn)


def _c_call(sp, cp, h, wn, wr, bn):
  return pl.pallas_call(
      _c_body,
      grid=(NBLK,),
      in_specs=[
          pl.BlockSpec((NC, NB, D), lambda i: (0, i, 0)),
          pl.BlockSpec((NC, NB, D), lambda i: (0, i, 0)),
          pl.BlockSpec((NC, NB, D), lambda i: (0, i, 0)),
          _full((H, H)),
          _full((H, H)),
          _full((1, H)),
      ],
      out_specs=[
          pl.BlockSpec((NB, H), lambda i: (i, 0)),
          _full((1, H)),
          _full((1, H)),
      ],
      out_shape=[
          jax.ShapeDtypeStruct((N, H), jnp.float32),
          jax.ShapeDtypeStruct((1, H), jnp.float32),
          jax.ShapeDtypeStruct((1, H), jnp.float32),
      ],
  )(sp, cp, h, wn, wr, bn)


def _d_call(pre, ssum, ssq, gamma, beta):
  return pl.pallas_call(
      _d_body,
      grid=(NBLK,),
      in_specs=[
          pl.BlockSpec((NB, H), lambda i: (i, 0)),
          _full((1, H)),
          _full((1, H)),
          _full((1, H)),
          _full((1, H)),
      ],
      out_specs=pl.BlockSpec((NC, NB, D), lambda i: (0, i, 0)),
      out_shape=jax.ShapeDtypeStruct((NC, N, D), jnp.float32),
  )(pre, ssum, ssq, gamma, beta)


def _pool_call(h, bidx, wc1, bc1, wc2, bc2):
  return pl.pallas_call(
      _pool_body,
      grid=(NBLK,),
      in_specs=[
          pl.BlockSpec((NC, NB, D), lambda i: (0, i, 0)),
          pl.BlockSpec((1, 1, NB), lambda i: (i, 0, 0)),
          _full((H, D)),
          _full((1, D)),
          _full((1, D)),
          _full((1, D)),
      ],
      out_specs=_full((G, D)),
      out_shape=jax.ShapeDtypeStruct((G, D), jnp.float32),
      scratch_shapes=[
          pltpu.VMEM((G, H), jnp.float32),
          pltpu.VMEM((G, D), jnp.float32),
      ],
  )(h, bidx, wc1, bc1, wc2, bc2)


def kernel(x, edge_index, batch, Wn0, bn0, Wr0, gamma0, beta0,
           Wn1, bn1, Wr1, gamma1, beta1, Wn2, bn2, Wr2, gamma2, beta2,
           Wc1, bc1, Wc2, bc2):
  src = edge_index[0]
  dst = edge_index[1]
  pad = E_PAD - E
  srcp = jnp.concatenate([src, jnp.zeros((pad,), jnp.int32)])
  dstp = jnp.concatenate([dst, jnp.full((pad,), N, jnp.int32)])

  nca = E_PAD // 32 // CH
  ncb = E_PAD // NS // CH
  src_a = srcp.reshape(32, nca, CH)
  dst_a = dstp.reshape(32, nca, CH)
  src_b = jnp.stack([srcp, srcp + N]).reshape(NC, NS, ncb, CH).reshape(32, ncb, CH)
  dst_b = jnp.broadcast_to(dstp.reshape(1, NS, ncb, CH),
                           (NC, NS, ncb, CH)).reshape(32, ncb, CH)
  zeros = jnp.zeros((RPT, D), jnp.float32)
  ones = jnp.ones((CH, D), jnp.float32)

  cntp = _cnt_kernel(dst_a, ones, zeros)
  s0p = _SEG_A(x, src_a, dst_a, zeros)

  bn0r = bn0.reshape(1, H)
  pre0, ss0, sq0 = _c0_call(s0p, cntp, x, Wn0, Wr0, bn0r)
  h1 = _d_call(pre0, ss0, sq0, gamma0.reshape(1, H), beta0.reshape(1, H))

  s1 = _SEG_B(h1.reshape(2 * N, D), src_b, dst_b, zeros)
  pre1, ss1, sq1 = _c_call(s1, cntp, h1, Wn1, Wr1, bn1.reshape(1, H))
  h2 = _d_call(pre1, ss1, sq1, gamma1.reshape(1, H), beta1.reshape(1, H))

  s2 = _SEG_B(h2.reshape(2 * N, D), src_b, dst_b, zeros)
  pre2, ss2, sq2 = _c_call(s2, cntp, h2, Wn2, Wr2, bn2.reshape(1, H))
  h3 = _d_call(pre2, ss2, sq2, gamma2.reshape(1, H), beta2.reshape(1, H))

  out128 = _pool_call(h3, batch.reshape(NBLK, 1, NB), Wc1, bc1.reshape(1, D),
                      Wc2.reshape(1, D), jnp.broadcast_to(bc2.reshape(1, 1), (1, D)))
  return out128[:, 0]


# speedup vs baseline: 3.6861x; 1.1898x over previous
import functools

import jax
import jax.numpy as jnp
from jax import lax
from jax.experimental import pallas as pl
from jax.experimental.pallas import tpu as pltpu
from jax.experimental.pallas import tpu_sc as plsc

N = 10000
E = 320000
D = 128
H = 256
G = 64

NC = 2
NS = 16
CH = 128
E_PAD = 327680
NACC = 10240
RPT = NACC // NS

NB = 1000
NBLK = N // NB

_MESH = plsc.VectorSubcoreMesh(core_axis_name="c", subcore_axis_name="s")


NCB = 40


def _make_segsum(nblocks):
  @functools.partial(
      pl.kernel,
      out_type=jax.ShapeDtypeStruct((NC, NACC, D), jnp.float32),
      mesh=_MESH,
      scratch_types=[
          pltpu.VMEM((NCB, CH), jnp.int32),
          pltpu.VMEM((NCB, CH), jnp.int32),
          pltpu.VMEM((CH, D), jnp.float32),
          pltpu.VMEM((CH, D), jnp.float32),
          pltpu.VMEM_SHARED((NACC, D), jnp.float32),
          pltpu.SemaphoreType.DMA,
          pltpu.SemaphoreType.DMA,
      ],
  )
  def seg(table_hbm, src_hbm, dst_hbm, zeros_hbm, out_hbm,
          srcv, dstv, rows0, rows1, acc, sem0, sem1):
    cid = lax.axis_index("c")
    sid = lax.axis_index("s")
    tid = cid * NS + sid
    pltpu.sync_copy(zeros_hbm, acc.at[pl.ds(sid * RPT, RPT)])
    plsc.subcore_barrier()

    def blk(b, carry):
      pltpu.sync_copy(src_hbm.at[tid, pl.ds(b * NCB, NCB)], srcv)
      pltpu.sync_copy(dst_hbm.at[tid, pl.ds(b * NCB, NCB)], dstv)
      pltpu.async_copy(table_hbm.at[srcv.at[0]], rows0, sem0)

      def pair(i, c):
        g0 = 2 * i
        pltpu.async_copy(table_hbm.at[srcv.at[g0 + 1]], rows1, sem1)
        pltpu.make_async_copy(table_hbm.at[pl.ds(0, CH)], rows0, sem0).wait()
        pltpu.sync_copy(rows0, acc.at[dstv.at[g0]], add=True)

        @pl.when(g0 + 2 < NCB)
        def _():
          pltpu.async_copy(table_hbm.at[srcv.at[g0 + 2]], rows0, sem0)

        pltpu.make_async_copy(table_hbm.at[pl.ds(0, CH)], rows1, sem1).wait()
        pltpu.sync_copy(rows1, acc.at[dstv.at[g0 + 1]], add=True)
        return c

      lax.fori_loop(0, NCB // 2, pair, 0)
      return carry

    lax.fori_loop(0, nblocks, blk, 0)
    plsc.subcore_barrier()
    pltpu.sync_copy(acc.at[pl.ds(sid * RPT, RPT)],
                    out_hbm.at[cid, pl.ds(sid * RPT, RPT)])

  return seg


_SEG_A = _make_segsum(2)
_SEG_B = _make_segsum(4)


@functools.partial(
    pl.kernel,
    out_type=jax.ShapeDtypeStruct((NC, NACC, D), jnp.float32),
    mesh=_MESH,
    scratch_types=[
        pltpu.VMEM((E_PAD // 32 // CH, CH), jnp.int32),
        pltpu.VMEM((CH, D), jnp.float32),
        pltpu.VMEM_SHARED((NACC, D), jnp.float32),
    ],
)
def _cnt_kernel(dst_hbm, ones_hbm, zeros_hbm, out_hbm, dstv, onesv, acc):
  cid = lax.axis_index("c")
  sid = lax.axis_index("s")
  tid = cid * NS + sid
  pltpu.sync_copy(zeros_hbm, acc.at[pl.ds(sid * RPT, RPT)])
  pltpu.sync_copy(ones_hbm, onesv)
  pltpu.sync_copy(dst_hbm.at[tid], dstv)
  plsc.subcore_barrier()

  def body(g, carry):
    pltpu.sync_copy(onesv, acc.at[dstv.at[g]], add=True)
    return carry

  lax.fori_loop(0, E_PAD // 32 // CH, body, 0)
  plsc.subcore_barrier()
  pltpu.sync_copy(acc.at[pl.ds(sid * RPT, RPT)],
                  out_hbm.at[cid, pl.ds(sid * RPT, RPT)])


def _dot(a, b):
  return jnp.dot(a, b, preferred_element_type=jnp.float32)


def _c0_body(sp, cp, x, wn, wr, bn, pre, ssum, ssq):
  pid = pl.program_id(0)
  inv = 1.0 / jnp.maximum(cp[0] + cp[1], 1.0)
  s = (sp[0] + sp[1]) * inv
  acc = _dot(s, wn[...]) + _dot(x[...], wr[...]) + bn[...]
  pre[...] = acc

  @pl.when(pid == 0)
  def _():
    ssum[...] = jnp.zeros_like(ssum)
    ssq[...] = jnp.zeros_like(ssq)

  ssum[...] += jnp.sum(acc, axis=0, keepdims=True)
  ssq[...] += jnp.sum(acc * acc, axis=0, keepdims=True)


def _c_body(sp, cp, h, wn, wr, bn, pre, ssum, ssq):
  pid = pl.program_id(0)
  inv = 1.0 / jnp.maximum(cp[0] + cp[1], 1.0)
  w_n = wn[...]
  w_r = wr[...]
  acc = (_dot(sp[0] * inv, w_n[0:D]) + _dot(sp[1] * inv, w_n[D:2 * D])
         + _dot(h[0], w_r[0:D]) + _dot(h[1], w_r[D:2 * D]) + bn[...])
  pre[...] = acc

  @pl.when(pid == 0)
  def _():
    ssum[...] = jnp.zeros_like(ssum)
    ssq[...] = jnp.zeros_like(ssq)

  ssum[...] += jnp.sum(acc, axis=0, keepdims=True)
  ssq[...] += jnp.sum(acc * acc, axis=0, keepdims=True)


def _d_body(pre, ssum, ssq, gamma, beta, h_out):
  mu = ssum[...] * (1.0 / N)
  var = ssq[...] * (1.0 / N) - mu * mu
  scale = gamma[...] * lax.rsqrt(var + 1e-5)
  shift = beta[...] - mu * scale
  v = jnp.maximum(pre[...] * scale + shift, 0.0)
  h_out[0] = v[:, 0:D]
  h_out[1] = v[:, D:2 * D]


def _pool_body(h, bidx, wc1, bc1, wc2, bc2, out, gacc, ccnt):
  pid = pl.program_id(0)

  @pl.when(pid == 0)
  def _():
    gacc[...] = jnp.zeros_like(gacc)
    ccnt[...] = jnp.zeros_like(ccnt)

  onehot = (lax.broadcasted_iota(jnp.int32, (G, NB), 0)
            == bidx[0]).astype(jnp.float32)
  gacc[:, 0:D] += _dot(onehot, h[0])
  gacc[:, D:2 * D] += _dot(onehot, h[1])
  ccnt[...] += jnp.sum(onehot, axis=1, keepdims=True)

  @pl.when(pid == NBLK - 1)
  def _():
    inv = 1.0 / jnp.maximum(ccnt[...], 1.0)
    g = gacc[...] * jnp.concatenate([inv, inv], axis=1)
    z = jnp.maximum(_dot(g, wc1[...]) + bc1[...], 0.0)
    out[...] = jnp.sum(z * wc2[...], axis=1, keepdims=True) + bc2[...]


def _full(shape):
  return pl.BlockSpec(shape, lambda i: tuple(0 for _ in shape))


def _c0_call(sp, cp, x, wn, wr, bn):
  return pl.pallas_call(
      _c0_body,
      grid=(NBLK,),
      in_specs=[
          pl.BlockSpec((NC, NB, D), lambda i: (0, i, 0)),
          pl.BlockSpec((NC, NB, D), lambda i: (0, i, 0)),
          pl.BlockSpec((NB, D), lambda i: (i, 0)),
          _full((D, H)),
          _full((D, H)),
          _full((1, H)),
      ],
      out_specs=[
          pl.BlockSpec((NB, H), lambda i: (i, 0)),
          _full((1, H)),
          _full((1, H)),
      ],
      out_shape=[
          jax.ShapeDtypeStruct((N, H), jnp.float32),
          jax.ShapeDtypeStruct((1, H), jnp.float32),
          jax.ShapeDtypeStruct((1, H), jnp.float32),
      ],
  )(sp, cp, x, wn, wr, bn)


def _c_call(sp, cp, h, wn, wr, bn):
  return pl.pallas_call(
      _c_body,
      grid=(NBLK,),
      in_specs=[
          pl.BlockSpec((NC, NB, D), lambda i: (0, i, 0)),
          pl.BlockSpec((NC, NB, D), lambda i: (0, i, 0)),
          pl.BlockSpec((NC, NB, D), lambda i: (0, i, 0)),
          _full((H, H)),
          _full((H, H)),
          _full((1, H)),
      ],
      out_specs=[
          pl.BlockSpec((NB, H), lambda i: (i, 0)),
          _full((1, H)),
          _full((1, H)),
      ],
      out_shape=[
          jax.ShapeDtypeStruct((N, H), jnp.float32),
          jax.ShapeDtypeStruct((1, H), jnp.float32),
          jax.ShapeDtypeStruct((1, H), jnp.float32),
      ],
  )(sp, cp, h, wn, wr, bn)


def _d_call(pre, ssum, ssq, gamma, beta):
  return pl.pallas_call(
      _d_body,
      grid=(NBLK,),
      in_specs=[
          pl.BlockSpec((NB, H), lambda i: (i, 0)),
          _full((1, H)),
          _full((1, H)),
          _full((1, H)),
          _full((1, H)),
      ],
      out_specs=pl.BlockSpec((NC, NB, D), lambda i: (0, i, 0)),
      out_shape=jax.ShapeDtypeStruct((NC, N, D), jnp.float32),
  )(pre, ssum, ssq, gamma, beta)


def _pool_call(h, bidx, wc1, bc1, wc2, bc2):
  return pl.pallas_call(
      _pool_body,
      grid=(NBLK,),
      in_specs=[
          pl.BlockSpec((NC, NB, D), lambda i: (0, i, 0)),
          pl.BlockSpec((1, 1, NB), lambda i: (i, 0, 0)),
          _full((H, D)),
          _full((1, D)),
          _full((1, D)),
          _full((1, D)),
      ],
      out_specs=_full((G, D)),
      out_shape=jax.ShapeDtypeStruct((G, D), jnp.float32),
      scratch_shapes=[
          pltpu.VMEM((G, H), jnp.float32),
          pltpu.VMEM((G, D), jnp.float32),
      ],
  )(h, bidx, wc1, bc1, wc2, bc2)


def kernel(x, edge_index, batch, Wn0, bn0, Wr0, gamma0, beta0,
           Wn1, bn1, Wr1, gamma1, beta1, Wn2, bn2, Wr2, gamma2, beta2,
           Wc1, bc1, Wc2, bc2):
  src = edge_index[0]
  dst = edge_index[1]
  pad = E_PAD - E
  srcp = jnp.concatenate([src, jnp.zeros((pad,), jnp.int32)])
  dstp = jnp.concatenate([dst, jnp.full((pad,), N, jnp.int32)])

  nca = E_PAD // 32 // CH
  ncb = E_PAD // NS // CH
  src_a = srcp.reshape(32, nca, CH)
  dst_a = dstp.reshape(32, nca, CH)
  src_b = jnp.stack([srcp, srcp + N]).reshape(NC, NS, ncb, CH).reshape(32, ncb, CH)
  dst_b = jnp.broadcast_to(dstp.reshape(1, NS, ncb, CH),
                           (NC, NS, ncb, CH)).reshape(32, ncb, CH)
  zeros = jnp.zeros((RPT, D), jnp.float32)
  ones = jnp.ones((CH, D), jnp.float32)

  cntp = _cnt_kernel(dst_a, ones, zeros)
  s0p = _SEG_A(x, src_a, dst_a, zeros)

  bn0r = bn0.reshape(1, H)
  pre0, ss0, sq0 = _c0_call(s0p, cntp, x, Wn0, Wr0, bn0r)
  h1 = _d_call(pre0, ss0, sq0, gamma0.reshape(1, H), beta0.reshape(1, H))

  s1 = _SEG_B(h1.reshape(2 * N, D), src_b, dst_b, zeros)
  pre1, ss1, sq1 = _c_call(s1, cntp, h1, Wn1, Wr1, bn1.reshape(1, H))
  h2 = _d_call(pre1, ss1, sq1, gamma1.reshape(1, H), beta1.reshape(1, H))

  s2 = _SEG_B(h2.reshape(2 * N, D), src_b, dst_b, zeros)
  pre2, ss2, sq2 = _c_call(s2, cntp, h2, Wn2, Wr2, bn2.reshape(1, H))
  h3 = _d_call(pre2, ss2, sq2, gamma2.reshape(1, H), beta2.reshape(1, H))

  out128 = _pool_call(h3, batch.reshape(NBLK, 1, NB), Wc1, bc1.reshape(1, D),
                      Wc2.reshape(1, D), jnp.broadcast_to(bc2.reshape(1, 1), (1, D)))
  return out128[:, 0]
